# baseline (speedup 1.0000x reference)
import dataclasses
import functools

import jax
import jax.numpy as jnp
from jax import lax
from jax.experimental import pallas as pl
from jax.experimental.pallas import tpu as pltpu
from jax.experimental.pallas import tpu_sc as plsc

N = 10000
E = 320000
F = 128
HID = 128
CLS = 64

NC, NS, LN = 2, 16, 16
NW = NC * NS
KE = 40
NB_E = 250
EPW = NB_E * KE
NBUF = 5
NPAD = 10240
ROWS_PT = NPAD // NS
ZROWS = KE
DEG_K = 25
UNROLL = 4


@functools.cache
def _mesh():
    return plsc.VectorSubcoreMesh(core_axis_name="c", subcore_axis_name="s")


@functools.cache
def _sc_params():
    cp = pltpu.CompilerParams()
    if "needs_layout_passes" in pltpu.CompilerParams.__dataclass_fields__:
        cp = dataclasses.replace(cp, needs_layout_passes=False)
    return cp


@functools.cache
def _make_sc_deg():
    @functools.partial(
        pl.kernel,
        mesh=_mesh(),
        compiler_params=_sc_params(),
        out_type=jax.ShapeDtypeStruct((NC, NPAD), jnp.float32),
        scratch_types=[
            pltpu.VMEM((NB_E, KE), jnp.int32),
            pltpu.VMEM((EPW,), jnp.float32),
            pltpu.VMEM((ROWS_PT,), jnp.float32),
            pltpu.VMEM_SHARED((NPAD,), jnp.float32),
            pltpu.SemaphoreType.DMA,
            pltpu.SemaphoreType.DMA,
        ],
    )
    def _sc_deg(col_hbm, ew_hbm, out_hbm, colb, ewb, zbuf, deg_sh, psem, dsem):
        c = lax.axis_index("c")
        s = lax.axis_index("s")
        wid = s * NC + c
        base = wid * EPW

        ep = pltpu.async_copy(ew_hbm.at[pl.ds(base, EPW)], ewb, psem)

        @pl.loop(0, NB_E, step=DEG_K)
        def _(g):
            @pl.loop(0, DEG_K)
            def _(j):
                b = g + j
                pltpu.async_copy(col_hbm.at[pl.ds(base + b * KE, KE)],
                                 colb.at[b], dsem)

            @pl.loop(0, DEG_K)
            def _(j):
                b = g + j
                pltpu.make_async_copy(col_hbm.at[pl.ds(base + b * KE, KE)],
                                      colb.at[b], dsem).wait()

        @pl.loop(0, ROWS_PT, step=LN)
        def _(i):
            zbuf[pl.ds(i, LN)] = jnp.zeros((LN,), jnp.float32)

        pltpu.sync_copy(zbuf, deg_sh.at[pl.ds(s * ROWS_PT, ROWS_PT)])
        ep.wait()
        plsc.subcore_barrier()

        @pl.loop(0, NB_E, step=DEG_K)
        def _(g):
            @pl.loop(0, DEG_K)
            def _(j):
                b = g + j
                pltpu.async_copy(ewb.at[pl.ds(b * KE, KE)],
                                 deg_sh.at[colb.at[b]], dsem, add=True)

            @pl.loop(0, DEG_K)
            def _(j):
                b = g + j
                pltpu.make_async_copy(ewb.at[pl.ds(b * KE, KE)],
                                      deg_sh.at[colb.at[b]], dsem).wait()

        plsc.subcore_barrier()
        pltpu.sync_copy(
            deg_sh.at[pl.ds(s * ROWS_PT, ROWS_PT)],
            out_hbm.at[c, pl.ds(s * ROWS_PT, ROWS_PT)],
        )

    return _sc_deg


@functools.cache
def _make_sc_scatter(width):
    nreg = width // LN

    @functools.partial(
        pl.kernel,
        mesh=_mesh(),
        compiler_params=_sc_params(),
        out_type=jax.ShapeDtypeStruct((NC, NPAD, width), jnp.float32),
        scratch_types=[
            pltpu.VMEM((NBUF, KE), jnp.int32),
            pltpu.VMEM((NBUF, KE), jnp.int32),
            pltpu.VMEM((NBUF, KE), jnp.float32),
            pltpu.VMEM((NBUF, KE, width), jnp.float32),
            pltpu.VMEM_SHARED((NPAD, width), jnp.float32),
            pltpu.SemaphoreType.DMA((NBUF,)),
            pltpu.SemaphoreType.DMA((NBUF,)),
            pltpu.SemaphoreType.DMA((NBUF,)),
        ],
    )
    def _k(row_hbm, col_hbm, ew_hbm, g_hbm, out_hbm,
           rowv, colv, ewv, msg, acc, isem, gsem, ssem):
        c = lax.axis_index("c")
        s = lax.axis_index("s")
        wid = s * NC + c

        @pl.loop(0, ZROWS)
        def _(i):
            for r in range(nreg):
                msg[0, i, pl.ds(r * LN, LN)] = jnp.zeros((LN,), jnp.float32)

        for t in range(ROWS_PT // ZROWS):
            pltpu.sync_copy(
                msg.at[0], acc.at[pl.ds(s * ROWS_PT + t * ZROWS, ZROWS)]
            )
        plsc.subcore_barrier()

        base = wid * EPW

        def issue_idx(b, j):
            off = base + b * KE
            pltpu.async_copy(row_hbm.at[pl.ds(off, KE)], rowv.at[j],
                             isem.at[j])
            pltpu.async_copy(col_hbm.at[pl.ds(off, KE)], colv.at[j],
                             isem.at[j])
            pltpu.async_copy(ew_hbm.at[pl.ds(off, KE)], ewv.at[j],
                             isem.at[j])

        def wait_idx(b, j):
            off = base + b * KE
            pltpu.make_async_copy(row_hbm.at[pl.ds(off, KE)], rowv.at[j],
                                  isem.at[j]).wait()
            pltpu.make_async_copy(col_hbm.at[pl.ds(off, KE)], colv.at[j],
                                  isem.at[j]).wait()
            pltpu.make_async_copy(ew_hbm.at[pl.ds(off, KE)], ewv.at[j],
                                  isem.at[j]).wait()

        def issue_gather(j):
            pltpu.async_copy(g_hbm.at[rowv.at[j]], msg.at[j], gsem.at[j])

        def wait_gather(j):
            pltpu.make_async_copy(g_hbm.at[rowv.at[j]], msg.at[j],
                                  gsem.at[j]).wait()

        def issue_scatter(j):
            pltpu.async_copy(msg.at[j], acc.at[colv.at[j]], ssem.at[j],
                             add=True)

        def wait_scatter(j):
            pltpu.make_async_copy(msg.at[j], acc.at[colv.at[j]],
                                  ssem.at[j]).wait()

        def scale(j):
            @plsc.parallel_loop(0, KE, 1, unroll=UNROLL)
            def _(e):
                w16 = plsc.load_gather(
                    ewv, [jnp.full((LN,), j, jnp.int32),
                          jnp.full((LN,), e, jnp.int32)])
                for r in range(nreg):
                    sl = pl.ds(r * LN, LN)
                    msg[j, e, sl] = msg[j, e, sl] * w16

        for j in range(NBUF):
            issue_idx(j, j)
        for j in range(NBUF):
            wait_idx(j, j)
            issue_gather(j)

        n_main = (NB_E - NBUF) // NBUF * NBUF

        @pl.loop(0, n_main, step=NBUF)
        def _(g):
            for j in range(NBUF):
                wait_gather(j)
                scale(j)
                issue_scatter(j)
            for j in range(NBUF):
                b2 = g + NBUF + j
                wait_scatter(j)
                issue_idx(b2, j)
            for j in range(NBUF):
                b2 = g + NBUF + j
                wait_idx(b2, j)
                issue_gather(j)

        for j in range(NBUF):
            wait_gather(j)
            scale(j)
            issue_scatter(j)
        for b in range(n_main + NBUF, NB_E):
            wait_scatter(0)
            issue_idx(b, 0)
            wait_idx(b, 0)
            issue_gather(0)
            wait_gather(0)
            scale(0)
            issue_scatter(0)
        for j in range(NBUF):
            wait_scatter(j)

        plsc.subcore_barrier()
        for t in range(ROWS_PT // ZROWS):
            pltpu.sync_copy(
                acc.at[pl.ds(s * ROWS_PT + t * ZROWS, ZROWS)],
                out_hbm.at[c, pl.ds(s * ROWS_PT + t * ZROWS, ZROWS)],
            )

    return _k


_BN = 400
_NBLK = N // _BN


def _dot(a, b):
    return lax.dot_general(
        a, b, (((1,), (0,)), ((), ())),
        preferred_element_type=jnp.float32,
        precision=lax.Precision.HIGHEST,
    )


def _tc_mm1disg(x, w1, d0, d1):
    def body(x_ref, w_ref, d0_ref, d1_ref, dis_ref, g_ref):
        h1 = _dot(x_ref[...], w_ref[...])
        dis = lax.rsqrt(1.0 + d0_ref[...] + d1_ref[...])
        dis_ref[...] = dis
        g_ref[...] = h1 * dis

    return pl.pallas_call(
        body,
        grid=(_NBLK,),
        in_specs=[
            pl.BlockSpec((_BN, F), lambda i: (i, 0)),
            pl.BlockSpec((F, HID), lambda i: (0, 0)),
            pl.BlockSpec((_BN, 1), lambda i: (i, 0)),
            pl.BlockSpec((_BN, 1), lambda i: (i, 0)),
        ],
        out_specs=[
            pl.BlockSpec((_BN, 1), lambda i: (i, 0)),
            pl.BlockSpec((_BN, HID), lambda i: (i, 0)),
        ],
        out_shape=[
            jax.ShapeDtypeStruct((N, 1), jnp.float32),
            jax.ShapeDtypeStruct((N, HID), jnp.float32),
        ],
    )(x, w1, d0, d1)


def _tc_mid(s1, g1, dis, b1, w2):
    def body(s_ref, g_ref, dis_ref, b_ref, w_ref, o_ref):
        ssum = s_ref[0] + s_ref[1] + g_ref[...]
        z = jax.nn.sigmoid(dis_ref[...] * ssum + b_ref[...])
        h2 = _dot(z, w_ref[...])
        o_ref[...] = h2 * dis_ref[...]

    return pl.pallas_call(
        body,
        grid=(_NBLK,),
        in_specs=[
            pl.BlockSpec((NC, _BN, HID), lambda i: (0, i, 0)),
            pl.BlockSpec((_BN, HID), lambda i: (i, 0)),
            pl.BlockSpec((_BN, 1), lambda i: (i, 0)),
            pl.BlockSpec((1, HID), lambda i: (0, 0)),
            pl.BlockSpec((HID, 128), lambda i: (0, 0)),
        ],
        out_specs=pl.BlockSpec((_BN, 128), lambda i: (i, 0)),
        out_shape=jax.ShapeDtypeStruct((N, 128), jnp.float32),
    )(s1, g1, dis, b1, w2)


def _tc_final(s2, g2, dis, b2):
    def body(s_ref, g_ref, dis_ref, b_ref, o_ref):
        ssum = (s_ref[0] + s_ref[1] + g_ref[...])[:, :CLS]
        o_ref[...] = jnp.tanh(dis_ref[...] * ssum + b_ref[...])

    return pl.pallas_call(
        body,
        grid=(_NBLK,),
        in_specs=[
            pl.BlockSpec((NC, _BN, 128), lambda i: (0, i, 0)),
            pl.BlockSpec((_BN, 128), lambda i: (i, 0)),
            pl.BlockSpec((_BN, 1), lambda i: (i, 0)),
            pl.BlockSpec((1, CLS), lambda i: (0, 0)),
        ],
        out_specs=pl.BlockSpec((_BN, CLS), lambda i: (i, 0)),
        out_shape=jax.ShapeDtypeStruct((N, CLS), jnp.float32),
    )(s2, g2, dis, b2)


def kernel(x, edge_index, edge_weight, W1, b1, W2, b2):
    row = edge_index[0].astype(jnp.int32)
    col = edge_index[1].astype(jnp.int32)
    ew = edge_weight.astype(jnp.float32)

    deg_p = _make_sc_deg()(col, ew)

    d0 = deg_p[0, :N].reshape(N, 1)
    d1 = deg_p[1, :N].reshape(N, 1)
    dis, g1 = _tc_mm1disg(x, W1, d0, d1)

    s1 = _make_sc_scatter(128)(row, col, ew, g1)

    w2p = jnp.concatenate(
        [W2, jnp.zeros((HID, 128 - CLS), jnp.float32)], axis=1
    )
    g2 = _tc_mid(s1, g1, dis, b1.reshape(1, HID), w2p)

    s2 = _make_sc_scatter(128)(row, col, ew, g2)

    return _tc_final(s2, g2, dis, b2.reshape(1, CLS))


# speedup vs baseline: 1.0247x; 1.0247x over previous
import dataclasses
import functools

import jax
import jax.numpy as jnp
from jax import lax
from jax.experimental import pallas as pl
from jax.experimental.pallas import tpu as pltpu
from jax.experimental.pallas import tpu_sc as plsc

N = 10000
E = 320000
F = 128
HID = 128
CLS = 64

NC, NS, LN = 2, 16, 16
NW = NC * NS
KE = 80
NB_E = 125
EPW = NB_E * KE
NBUF = 4
NPAD = 10240
ROWS_PT = NPAD // NS
ZROWS = KE
DEG_K = 25
UNROLL = 8


@functools.cache
def _mesh():
    return plsc.VectorSubcoreMesh(core_axis_name="c", subcore_axis_name="s")


@functools.cache
def _sc_params():
    cp = pltpu.CompilerParams()
    if "needs_layout_passes" in pltpu.CompilerParams.__dataclass_fields__:
        cp = dataclasses.replace(cp, needs_layout_passes=False)
    return cp


@functools.cache
def _make_sc_deg():
    @functools.partial(
        pl.kernel,
        mesh=_mesh(),
        compiler_params=_sc_params(),
        out_type=jax.ShapeDtypeStruct((NC, NPAD), jnp.float32),
        scratch_types=[
            pltpu.VMEM((NB_E, KE), jnp.int32),
            pltpu.VMEM((EPW,), jnp.float32),
            pltpu.VMEM((ROWS_PT,), jnp.float32),
            pltpu.VMEM_SHARED((NPAD,), jnp.float32),
            pltpu.SemaphoreType.DMA,
            pltpu.SemaphoreType.DMA,
        ],
    )
    def _sc_deg(col_hbm, ew_hbm, out_hbm, colb, ewb, zbuf, deg_sh, psem, dsem):
        c = lax.axis_index("c")
        s = lax.axis_index("s")
        wid = s * NC + c
        base = wid * EPW

        ep = pltpu.async_copy(ew_hbm.at[pl.ds(base, EPW)], ewb, psem)

        @pl.loop(0, NB_E, step=DEG_K)
        def _(g):
            @pl.loop(0, DEG_K)
            def _(j):
                b = g + j
                pltpu.async_copy(col_hbm.at[pl.ds(base + b * KE, KE)],
                                 colb.at[b], dsem)

            @pl.loop(0, DEG_K)
            def _(j):
                b = g + j
                pltpu.make_async_copy(col_hbm.at[pl.ds(base + b * KE, KE)],
                                      colb.at[b], dsem).wait()

        @pl.loop(0, ROWS_PT, step=LN)
        def _(i):
            zbuf[pl.ds(i, LN)] = jnp.zeros((LN,), jnp.float32)

        pltpu.sync_copy(zbuf, deg_sh.at[pl.ds(s * ROWS_PT, ROWS_PT)])
        ep.wait()
        plsc.subcore_barrier()

        @pl.loop(0, NB_E, step=DEG_K)
        def _(g):
            @pl.loop(0, DEG_K)
            def _(j):
                b = g + j
                pltpu.async_copy(ewb.at[pl.ds(b * KE, KE)],
                                 deg_sh.at[colb.at[b]], dsem, add=True)

            @pl.loop(0, DEG_K)
            def _(j):
                b = g + j
                pltpu.make_async_copy(ewb.at[pl.ds(b * KE, KE)],
                                      deg_sh.at[colb.at[b]], dsem).wait()

        plsc.subcore_barrier()
        pltpu.sync_copy(
            deg_sh.at[pl.ds(s * ROWS_PT, ROWS_PT)],
            out_hbm.at[c, pl.ds(s * ROWS_PT, ROWS_PT)],
        )

    return _sc_deg


@functools.cache
def _make_sc_scatter(width):
    nreg = width // LN

    @functools.partial(
        pl.kernel,
        mesh=_mesh(),
        compiler_params=_sc_params(),
        out_type=jax.ShapeDtypeStruct((NC, NPAD, width), jnp.float32),
        scratch_types=[
            pltpu.VMEM((NBUF, 2, KE), jnp.int32),
            pltpu.VMEM((NBUF, KE), jnp.float32),
            pltpu.VMEM((NBUF, KE, width), jnp.float32),
            pltpu.VMEM_SHARED((NPAD, width), jnp.float32),
            pltpu.SemaphoreType.DMA((NBUF,)),
            pltpu.SemaphoreType.DMA((NBUF,)),
            pltpu.SemaphoreType.DMA((NBUF,)),
        ],
    )
    def _k(ep_hbm, ew_hbm, g_hbm, out_hbm, epk, ewv, msg, acc,
           isem, gsem, ssem):
        c = lax.axis_index("c")
        s = lax.axis_index("s")
        wid = s * NC + c
        base = wid * EPW

        @pl.loop(0, ZROWS)
        def _(i):
            for r in range(nreg):
                msg[0, i, pl.ds(r * LN, LN)] = jnp.zeros((LN,), jnp.float32)

        for t in range(ROWS_PT // ZROWS):
            pltpu.sync_copy(
                msg.at[0], acc.at[pl.ds(s * ROWS_PT + t * ZROWS, ZROWS)]
            )
        plsc.subcore_barrier()

        def issue_idx(b, j):
            pltpu.async_copy(ep_hbm.at[wid, b], epk.at[j], isem.at[j])
            pltpu.async_copy(ew_hbm.at[pl.ds(base + b * KE, KE)], ewv.at[j],
                             isem.at[j])

        def wait_idx(b, j):
            pltpu.make_async_copy(ep_hbm.at[wid, b], epk.at[j],
                                  isem.at[j]).wait()
            pltpu.make_async_copy(ew_hbm.at[pl.ds(base + b * KE, KE)],
                                  ewv.at[j], isem.at[j]).wait()

        def issue_gather(j):
            pltpu.async_copy(g_hbm.at[epk.at[j, 0]], msg.at[j], gsem.at[j])

        def wait_gather(j):
            pltpu.make_async_copy(g_hbm.at[epk.at[j, 0]], msg.at[j],
                                  gsem.at[j]).wait()

        def issue_scatter(j):
            pltpu.async_copy(msg.at[j], acc.at[epk.at[j, 1]], ssem.at[j],
                             add=True)

        def wait_scatter(j):
            pltpu.make_async_copy(msg.at[j], acc.at[epk.at[j, 1]],
                                  ssem.at[j]).wait()

        def scale(j):
            @plsc.parallel_loop(0, KE, 1, unroll=UNROLL)
            def _(e):
                w16 = plsc.load_gather(
                    ewv, [jnp.full((LN,), j, jnp.int32),
                          jnp.full((LN,), e, jnp.int32)])
                for r in range(nreg):
                    sl = pl.ds(r * LN, LN)
                    msg[j, e, sl] = msg[j, e, sl] * w16

        for j in range(NBUF):
            issue_idx(j, j)
        for j in range(NBUF):
            wait_idx(j, j)
            issue_gather(j)

        n_main = (NB_E - NBUF) // NBUF * NBUF

        @pl.loop(0, n_main, step=NBUF)
        def _(g):
            for j in range(NBUF):
                wait_gather(j)
                scale(j)
                issue_scatter(j)
            for j in range(NBUF):
                b2 = g + NBUF + j
                wait_scatter(j)
                issue_idx(b2, j)
            for j in range(NBUF):
                b2 = g + NBUF + j
                wait_idx(b2, j)
                issue_gather(j)

        for j in range(NBUF):
            wait_gather(j)
            scale(j)
            issue_scatter(j)
        for b in range(n_main + NBUF, NB_E):
            wait_scatter(0)
            issue_idx(b, 0)
            wait_idx(b, 0)
            issue_gather(0)
            wait_gather(0)
            scale(0)
            issue_scatter(0)
        for j in range(NBUF):
            wait_scatter(j)

        plsc.subcore_barrier()
        for t in range(ROWS_PT // ZROWS):
            pltpu.sync_copy(
                acc.at[pl.ds(s * ROWS_PT + t * ZROWS, ZROWS)],
                out_hbm.at[c, pl.ds(s * ROWS_PT + t * ZROWS, ZROWS)],
            )

    return _k


_BN = 400
_NBLK = N // _BN


def _dot(a, b):
    return lax.dot_general(
        a, b, (((1,), (0,)), ((), ())),
        preferred_element_type=jnp.float32,
        precision=lax.Precision.HIGHEST,
    )


def _tc_mm1disg(x, w1, d0, d1):
    def body(x_ref, w_ref, d0_ref, d1_ref, dis_ref, g_ref):
        h1 = _dot(x_ref[...], w_ref[...])
        dis = lax.rsqrt(1.0 + d0_ref[...] + d1_ref[...])
        dis_ref[...] = dis
        g_ref[...] = h1 * dis

    return pl.pallas_call(
        body,
        grid=(_NBLK,),
        in_specs=[
            pl.BlockSpec((_BN, F), lambda i: (i, 0)),
            pl.BlockSpec((F, HID), lambda i: (0, 0)),
            pl.BlockSpec((_BN, 1), lambda i: (i, 0)),
            pl.BlockSpec((_BN, 1), lambda i: (i, 0)),
        ],
        out_specs=[
            pl.BlockSpec((_BN, 1), lambda i: (i, 0)),
            pl.BlockSpec((_BN, HID), lambda i: (i, 0)),
        ],
        out_shape=[
            jax.ShapeDtypeStruct((N, 1), jnp.float32),
            jax.ShapeDtypeStruct((N, HID), jnp.float32),
        ],
    )(x, w1, d0, d1)


def _tc_mid(s1, g1, dis, b1, w2):
    def body(s_ref, g_ref, dis_ref, b_ref, w_ref, o_ref):
        ssum = s_ref[0] + s_ref[1] + g_ref[...]
        z = jax.nn.sigmoid(dis_ref[...] * ssum + b_ref[...])
        h2 = _dot(z, w_ref[...])
        o_ref[...] = h2 * dis_ref[...]

    return pl.pallas_call(
        body,
        grid=(_NBLK,),
        in_specs=[
            pl.BlockSpec((NC, _BN, HID), lambda i: (0, i, 0)),
            pl.BlockSpec((_BN, HID), lambda i: (i, 0)),
            pl.BlockSpec((_BN, 1), lambda i: (i, 0)),
            pl.BlockSpec((1, HID), lambda i: (0, 0)),
            pl.BlockSpec((HID, 128), lambda i: (0, 0)),
        ],
        out_specs=pl.BlockSpec((_BN, 128), lambda i: (i, 0)),
        out_shape=jax.ShapeDtypeStruct((N, 128), jnp.float32),
    )(s1, g1, dis, b1, w2)


def _tc_final(s2, g2, dis, b2):
    def body(s_ref, g_ref, dis_ref, b_ref, o_ref):
        ssum = (s_ref[0] + s_ref[1] + g_ref[...])[:, :CLS]
        o_ref[...] = jnp.tanh(dis_ref[...] * ssum + b_ref[...])

    return pl.pallas_call(
        body,
        grid=(_NBLK,),
        in_specs=[
            pl.BlockSpec((NC, _BN, 128), lambda i: (0, i, 0)),
            pl.BlockSpec((_BN, 128), lambda i: (i, 0)),
            pl.BlockSpec((_BN, 1), lambda i: (i, 0)),
            pl.BlockSpec((1, CLS), lambda i: (0, 0)),
        ],
        out_specs=pl.BlockSpec((_BN, CLS), lambda i: (i, 0)),
        out_shape=jax.ShapeDtypeStruct((N, CLS), jnp.float32),
    )(s2, g2, dis, b2)


def kernel(x, edge_index, edge_weight, W1, b1, W2, b2):
    row = edge_index[0].astype(jnp.int32)
    col = edge_index[1].astype(jnp.int32)
    ew = edge_weight.astype(jnp.float32)

    epack = jnp.stack(
        [row.reshape(NW, NB_E, KE), col.reshape(NW, NB_E, KE)], axis=2)

    deg_p = _make_sc_deg()(col, ew)

    d0 = deg_p[0, :N].reshape(N, 1)
    d1 = deg_p[1, :N].reshape(N, 1)
    dis, g1 = _tc_mm1disg(x, W1, d0, d1)

    s1 = _make_sc_scatter(128)(epack, ew, g1)

    w2p = jnp.concatenate(
        [W2, jnp.zeros((HID, 128 - CLS), jnp.float32)], axis=1
    )
    g2 = _tc_mid(s1, g1, dis, b1.reshape(1, HID), w2p)

    s2 = _make_sc_scatter(128)(epack, ew, g2)

    return _tc_final(s2, g2, dis, b2.reshape(1, CLS))


# speedup vs baseline: 1.0320x; 1.0072x over previous
import dataclasses
import functools

import jax
import jax.numpy as jnp
from jax import lax
from jax.experimental import pallas as pl
from jax.experimental.pallas import tpu as pltpu
from jax.experimental.pallas import tpu_sc as plsc

N = 10000
E = 320000
F = 128
HID = 128
CLS = 64

NC, NS, LN = 2, 16, 16
NW = NC * NS
KE = 80
NB_E = 125
EPW = NB_E * KE
NBUF = 4
NPAD = 10240
ROWS_PT = NPAD // NS
ZROWS = KE
DEG_K = 25
UNROLL = 4


@functools.cache
def _mesh():
    return plsc.VectorSubcoreMesh(core_axis_name="c", subcore_axis_name="s")


@functools.cache
def _sc_params():
    cp = pltpu.CompilerParams()
    if "needs_layout_passes" in pltpu.CompilerParams.__dataclass_fields__:
        cp = dataclasses.replace(cp, needs_layout_passes=False)
    return cp


@functools.cache
def _make_sc_deg():
    @functools.partial(
        pl.kernel,
        mesh=_mesh(),
        compiler_params=_sc_params(),
        out_type=jax.ShapeDtypeStruct((NC, NPAD), jnp.float32),
        scratch_types=[
            pltpu.VMEM((NB_E, KE), jnp.int32),
            pltpu.VMEM((EPW,), jnp.float32),
            pltpu.VMEM((ROWS_PT,), jnp.float32),
            pltpu.VMEM_SHARED((NPAD,), jnp.float32),
            pltpu.SemaphoreType.DMA,
            pltpu.SemaphoreType.DMA,
        ],
    )
    def _sc_deg(col_hbm, ew_hbm, out_hbm, colb, ewb, zbuf, deg_sh, psem, dsem):
        c = lax.axis_index("c")
        s = lax.axis_index("s")
        wid = s * NC + c
        base = wid * EPW

        ep = pltpu.async_copy(ew_hbm.at[pl.ds(base, EPW)], ewb, psem)

        @pl.loop(0, NB_E, step=DEG_K)
        def _(g):
            @pl.loop(0, DEG_K)
            def _(j):
                b = g + j
                pltpu.async_copy(col_hbm.at[pl.ds(base + b * KE, KE)],
                                 colb.at[b], dsem)

            @pl.loop(0, DEG_K)
            def _(j):
                b = g + j
                pltpu.make_async_copy(col_hbm.at[pl.ds(base + b * KE, KE)],
                                      colb.at[b], dsem).wait()

        @pl.loop(0, ROWS_PT, step=LN)
        def _(i):
            zbuf[pl.ds(i, LN)] = jnp.zeros((LN,), jnp.float32)

        pltpu.sync_copy(zbuf, deg_sh.at[pl.ds(s * ROWS_PT, ROWS_PT)])
        ep.wait()
        plsc.subcore_barrier()

        @pl.loop(0, NB_E, step=DEG_K)
        def _(g):
            @pl.loop(0, DEG_K)
            def _(j):
                b = g + j
                pltpu.async_copy(ewb.at[pl.ds(b * KE, KE)],
                                 deg_sh.at[colb.at[b]], dsem, add=True)

            @pl.loop(0, DEG_K)
            def _(j):
                b = g + j
                pltpu.make_async_copy(ewb.at[pl.ds(b * KE, KE)],
                                      deg_sh.at[colb.at[b]], dsem).wait()

        plsc.subcore_barrier()
        pltpu.sync_copy(
            deg_sh.at[pl.ds(s * ROWS_PT, ROWS_PT)],
            out_hbm.at[c, pl.ds(s * ROWS_PT, ROWS_PT)],
        )

    return _sc_deg


@functools.cache
def _make_sc_scatter(width):
    nreg = width // LN

    @functools.partial(
        pl.kernel,
        mesh=_mesh(),
        compiler_params=_sc_params(),
        out_type=jax.ShapeDtypeStruct((NC, NPAD, width), jnp.float32),
        scratch_types=[
            pltpu.VMEM((NBUF, 2, KE), jnp.int32),
            pltpu.VMEM((NBUF, KE), jnp.float32),
            pltpu.VMEM((NBUF, KE, width), jnp.float32),
            pltpu.VMEM_SHARED((NPAD, width), jnp.float32),
            pltpu.SemaphoreType.DMA((NBUF,)),
            pltpu.SemaphoreType.DMA((NBUF,)),
            pltpu.SemaphoreType.DMA((NBUF,)),
        ],
    )
    def _k(ep_hbm, ew_hbm, g_hbm, out_hbm, epk, ewv, msg, acc,
           isem, gsem, ssem):
        c = lax.axis_index("c")
        s = lax.axis_index("s")
        wid = s * NC + c
        base = wid * EPW

        @pl.loop(0, ZROWS)
        def _(i):
            for r in range(nreg):
                msg[0, i, pl.ds(r * LN, LN)] = jnp.zeros((LN,), jnp.float32)

        for t in range(ROWS_PT // ZROWS):
            pltpu.sync_copy(
                msg.at[0], acc.at[pl.ds(s * ROWS_PT + t * ZROWS, ZROWS)]
            )
        plsc.subcore_barrier()

        def issue_idx(b, j):
            pltpu.async_copy(ep_hbm.at[wid, b], epk.at[j], isem.at[j])
            pltpu.async_copy(ew_hbm.at[pl.ds(base + b * KE, KE)], ewv.at[j],
                             isem.at[j])

        def wait_idx(b, j):
            pltpu.make_async_copy(ep_hbm.at[wid, b], epk.at[j],
                                  isem.at[j]).wait()
            pltpu.make_async_copy(ew_hbm.at[pl.ds(base + b * KE, KE)],
                                  ewv.at[j], isem.at[j]).wait()

        def issue_gather(j):
            pltpu.async_copy(g_hbm.at[epk.at[j, 0]], msg.at[j], gsem.at[j])

        def wait_gather(j):
            pltpu.make_async_copy(g_hbm.at[epk.at[j, 0]], msg.at[j],
                                  gsem.at[j]).wait()

        def issue_scatter(j):
            pltpu.async_copy(msg.at[j], acc.at[epk.at[j, 1]], ssem.at[j],
                             add=True)

        def wait_scatter(j):
            pltpu.make_async_copy(msg.at[j], acc.at[epk.at[j, 1]],
                                  ssem.at[j]).wait()

        def scale(j):
            @plsc.parallel_loop(0, KE, 1, unroll=UNROLL)
            def _(e):
                w16 = plsc.load_gather(
                    ewv, [jnp.full((LN,), j, jnp.int32),
                          jnp.full((LN,), e, jnp.int32)])
                for r in range(nreg):
                    sl = pl.ds(r * LN, LN)
                    msg[j, e, sl] = msg[j, e, sl] * w16

        for j in range(NBUF):
            issue_idx(j, j)
        for j in range(NBUF):
            wait_idx(j, j)
            issue_gather(j)

        n_main = (NB_E - NBUF) // NBUF * NBUF

        @pl.loop(0, n_main, step=NBUF)
        def _(g):
            for j in range(NBUF):
                wait_gather(j)
                scale(j)
                issue_scatter(j)
            for j in range(NBUF):
                b2 = g + NBUF + j
                wait_scatter(j)
                issue_idx(b2, j)
            for j in range(NBUF):
                b2 = g + NBUF + j
                wait_idx(b2, j)
                issue_gather(j)

        for j in range(NBUF):
            wait_gather(j)
            scale(j)
            issue_scatter(j)
        for b in range(n_main + NBUF, NB_E):
            wait_scatter(0)
            issue_idx(b, 0)
            wait_idx(b, 0)
            issue_gather(0)
            wait_gather(0)
            scale(0)
            issue_scatter(0)
        for j in range(NBUF):
            wait_scatter(j)

        plsc.subcore_barrier()
        for t in range(ROWS_PT // ZROWS):
            pltpu.sync_copy(
                acc.at[pl.ds(s * ROWS_PT + t * ZROWS, ZROWS)],
                out_hbm.at[c, pl.ds(s * ROWS_PT + t * ZROWS, ZROWS)],
            )

    return _k


_BN = 400
_NBLK = N // _BN


def _dot(a, b):
    return lax.dot_general(
        a, b, (((1,), (0,)), ((), ())),
        preferred_element_type=jnp.float32,
        precision=lax.Precision.HIGHEST,
    )


def _tc_mm1disg(x, w1, d0, d1):
    def body(x_ref, w_ref, d0_ref, d1_ref, dis_ref, g_ref):
        h1 = _dot(x_ref[...], w_ref[...])
        dis = lax.rsqrt(1.0 + d0_ref[...] + d1_ref[...])
        dis_ref[...] = dis
        g_ref[...] = h1 * dis

    return pl.pallas_call(
        body,
        grid=(_NBLK,),
        in_specs=[
            pl.BlockSpec((_BN, F), lambda i: (i, 0)),
            pl.BlockSpec((F, HID), lambda i: (0, 0)),
            pl.BlockSpec((_BN, 1), lambda i: (i, 0)),
            pl.BlockSpec((_BN, 1), lambda i: (i, 0)),
        ],
        out_specs=[
            pl.BlockSpec((_BN, 1), lambda i: (i, 0)),
            pl.BlockSpec((_BN, HID), lambda i: (i, 0)),
        ],
        out_shape=[
            jax.ShapeDtypeStruct((N, 1), jnp.float32),
            jax.ShapeDtypeStruct((N, HID), jnp.float32),
        ],
    )(x, w1, d0, d1)


def _tc_mid(s1, g1, dis, b1, w2):
    def body(s_ref, g_ref, dis_ref, b_ref, w_ref, o_ref):
        ssum = s_ref[0] + s_ref[1] + g_ref[...]
        z = jax.nn.sigmoid(dis_ref[...] * ssum + b_ref[...])
        h2 = _dot(z, w_ref[...])
        o_ref[...] = h2 * dis_ref[...]

    return pl.pallas_call(
        body,
        grid=(_NBLK,),
        in_specs=[
            pl.BlockSpec((NC, _BN, HID), lambda i: (0, i, 0)),
            pl.BlockSpec((_BN, HID), lambda i: (i, 0)),
            pl.BlockSpec((_BN, 1), lambda i: (i, 0)),
            pl.BlockSpec((1, HID), lambda i: (0, 0)),
            pl.BlockSpec((HID, 128), lambda i: (0, 0)),
        ],
        out_specs=pl.BlockSpec((_BN, 128), lambda i: (i, 0)),
        out_shape=jax.ShapeDtypeStruct((N, 128), jnp.float32),
    )(s1, g1, dis, b1, w2)


def _tc_final(s2, g2, dis, b2):
    def body(s_ref, g_ref, dis_ref, b_ref, o_ref):
        ssum = (s_ref[0] + s_ref[1] + g_ref[...])[:, :CLS]
        o_ref[...] = jnp.tanh(dis_ref[...] * ssum + b_ref[...])

    return pl.pallas_call(
        body,
        grid=(_NBLK,),
        in_specs=[
            pl.BlockSpec((NC, _BN, 128), lambda i: (0, i, 0)),
            pl.BlockSpec((_BN, 128), lambda i: (i, 0)),
            pl.BlockSpec((_BN, 1), lambda i: (i, 0)),
            pl.BlockSpec((1, CLS), lambda i: (0, 0)),
        ],
        out_specs=pl.BlockSpec((_BN, CLS), lambda i: (i, 0)),
        out_shape=jax.ShapeDtypeStruct((N, CLS), jnp.float32),
    )(s2, g2, dis, b2)


def kernel(x, edge_index, edge_weight, W1, b1, W2, b2):
    row = edge_index[0].astype(jnp.int32)
    col = edge_index[1].astype(jnp.int32)
    ew = edge_weight.astype(jnp.float32)

    epack = jnp.stack(
        [row.reshape(NW, NB_E, KE), col.reshape(NW, NB_E, KE)], axis=2)

    deg_p = _make_sc_deg()(col, ew)

    d0 = deg_p[0, :N].reshape(N, 1)
    d1 = deg_p[1, :N].reshape(N, 1)
    dis, g1 = _tc_mm1disg(x, W1, d0, d1)

    s1 = _make_sc_scatter(128)(epack, ew, g1)

    w2p = jnp.concatenate(
        [W2, jnp.zeros((HID, 128 - CLS), jnp.float32)], axis=1
    )
    g2 = _tc_mid(s1, g1, dis, b1.reshape(1, HID), w2p)

    s2 = _make_sc_scatter(128)(epack, ew, g2)

    return _tc_final(s2, g2, dis, b2.reshape(1, CLS))


# speedup vs baseline: 1.0560x; 1.0232x over previous
import dataclasses
import functools

import jax
import jax.numpy as jnp
from jax import lax
from jax.experimental import pallas as pl
from jax.experimental.pallas import tpu as pltpu
from jax.experimental.pallas import tpu_sc as plsc

N = 10000
E = 320000
F = 128
HID = 128
CLS = 64

NC, NS, LN = 2, 16, 16
NW = NC * NS
KE = 80
NB_E = 125
EPW = NB_E * KE
NBUF = 4
NPAD = 10240
ROWS_PT = NPAD // NS
ZROWS = KE
DEG_K = 25
UNROLL = 4


@functools.cache
def _mesh():
    return plsc.VectorSubcoreMesh(core_axis_name="c", subcore_axis_name="s")


@functools.cache
def _sc_params():
    cp = pltpu.CompilerParams()
    if "needs_layout_passes" in pltpu.CompilerParams.__dataclass_fields__:
        cp = dataclasses.replace(cp, needs_layout_passes=False)
    return cp


@functools.cache
def _make_sc_deg():
    @functools.partial(
        pl.kernel,
        mesh=_mesh(),
        compiler_params=_sc_params(),
        out_type=jax.ShapeDtypeStruct((NC, NPAD), jnp.float32),
        scratch_types=[
            pltpu.VMEM((NB_E, KE), jnp.int32),
            pltpu.VMEM((EPW,), jnp.float32),
            pltpu.VMEM((ROWS_PT,), jnp.float32),
            pltpu.VMEM_SHARED((NPAD,), jnp.float32),
            pltpu.SemaphoreType.DMA,
            pltpu.SemaphoreType.DMA,
        ],
    )
    def _sc_deg(col_hbm, ew_hbm, out_hbm, colb, ewb, zbuf, deg_sh, psem, dsem):
        c = lax.axis_index("c")
        s = lax.axis_index("s")
        wid = s * NC + c
        base = wid * EPW

        ep = pltpu.async_copy(ew_hbm.at[pl.ds(base, EPW)], ewb, psem)

        @pl.loop(0, NB_E, step=DEG_K)
        def _(g):
            @pl.loop(0, DEG_K)
            def _(j):
                b = g + j
                pltpu.async_copy(col_hbm.at[pl.ds(base + b * KE, KE)],
                                 colb.at[b], dsem)

            @pl.loop(0, DEG_K)
            def _(j):
                b = g + j
                pltpu.make_async_copy(col_hbm.at[pl.ds(base + b * KE, KE)],
                                      colb.at[b], dsem).wait()

        @pl.loop(0, ROWS_PT, step=LN)
        def _(i):
            zbuf[pl.ds(i, LN)] = jnp.zeros((LN,), jnp.float32)

        pltpu.sync_copy(zbuf, deg_sh.at[pl.ds(s * ROWS_PT, ROWS_PT)])
        ep.wait()
        plsc.subcore_barrier()

        @pl.loop(0, NB_E, step=DEG_K)
        def _(g):
            @pl.loop(0, DEG_K)
            def _(j):
                b = g + j
                pltpu.async_copy(ewb.at[pl.ds(b * KE, KE)],
                                 deg_sh.at[colb.at[b]], dsem, add=True)

            @pl.loop(0, DEG_K)
            def _(j):
                b = g + j
                pltpu.make_async_copy(ewb.at[pl.ds(b * KE, KE)],
                                      deg_sh.at[colb.at[b]], dsem).wait()

        plsc.subcore_barrier()
        pltpu.sync_copy(
            deg_sh.at[pl.ds(s * ROWS_PT, ROWS_PT)],
            out_hbm.at[c, pl.ds(s * ROWS_PT, ROWS_PT)],
        )

    return _sc_deg


@functools.cache
def _make_sc_scatter(gwidth, swidth):
    nreg = swidth // LN
    compact = swidth != gwidth

    scratch = [
        pltpu.VMEM((NBUF, KE), jnp.int32),
        pltpu.VMEM((NBUF, KE), jnp.int32),
        pltpu.VMEM((NBUF, KE), jnp.float32),
        pltpu.VMEM((NBUF, KE, gwidth), jnp.float32),
    ]
    if compact:
        scratch.append(pltpu.VMEM((NBUF, KE, swidth), jnp.float32))
    scratch += [
        pltpu.VMEM_SHARED((NPAD, swidth), jnp.float32),
        pltpu.SemaphoreType.DMA((NBUF,)),
        pltpu.SemaphoreType.DMA((NBUF,)),
        pltpu.SemaphoreType.DMA((NBUF,)),
    ]

    @functools.partial(
        pl.kernel,
        mesh=_mesh(),
        compiler_params=_sc_params(),
        out_type=jax.ShapeDtypeStruct((NC, NPAD, swidth), jnp.float32),
        scratch_types=scratch,
    )
    def _k(row_hbm, col_hbm, ew_hbm, g_hbm, out_hbm, rowv, colv, ewv, msg,
           *rest):
        if compact:
            smsg, acc, isem, gsem, ssem = rest
        else:
            acc, isem, gsem, ssem = rest
            smsg = msg
        c = lax.axis_index("c")
        s = lax.axis_index("s")
        wid = s * NC + c
        base = wid * EPW

        @pl.loop(0, ZROWS)
        def _(i):
            for r in range(nreg):
                smsg[0, i, pl.ds(r * LN, LN)] = jnp.zeros((LN,), jnp.float32)

        for t in range(ROWS_PT // ZROWS):
            pltpu.sync_copy(
                smsg.at[0], acc.at[pl.ds(s * ROWS_PT + t * ZROWS, ZROWS)]
            )
        plsc.subcore_barrier()

        def issue_idx(b, j):
            off = base + b * KE
            pltpu.async_copy(row_hbm.at[pl.ds(off, KE)], rowv.at[j],
                             isem.at[j])
            pltpu.async_copy(col_hbm.at[pl.ds(off, KE)], colv.at[j],
                             isem.at[j])
            pltpu.async_copy(ew_hbm.at[pl.ds(off, KE)], ewv.at[j],
                             isem.at[j])

        def wait_idx(b, j):
            off = base + b * KE
            pltpu.make_async_copy(row_hbm.at[pl.ds(off, KE)], rowv.at[j],
                                  isem.at[j]).wait()
            pltpu.make_async_copy(col_hbm.at[pl.ds(off, KE)], colv.at[j],
                                  isem.at[j]).wait()
            pltpu.make_async_copy(ew_hbm.at[pl.ds(off, KE)], ewv.at[j],
                                  isem.at[j]).wait()

        def issue_gather(j):
            pltpu.async_copy(g_hbm.at[rowv.at[j]], msg.at[j], gsem.at[j])

        def wait_gather(j):
            pltpu.make_async_copy(g_hbm.at[rowv.at[j]], msg.at[j],
                                  gsem.at[j]).wait()

        def issue_scatter(j):
            pltpu.async_copy(smsg.at[j], acc.at[colv.at[j]], ssem.at[j],
                             add=True)

        def wait_scatter(j):
            pltpu.make_async_copy(smsg.at[j], acc.at[colv.at[j]],
                                  ssem.at[j]).wait()

        def scale(j):
            @plsc.parallel_loop(0, KE, 1, unroll=UNROLL)
            def _(e):
                w16 = plsc.load_gather(
                    ewv, [jnp.full((LN,), j, jnp.int32),
                          jnp.full((LN,), e, jnp.int32)])
                for r in range(nreg):
                    sl = pl.ds(r * LN, LN)
                    smsg[j, e, sl] = msg[j, e, sl] * w16

        for j in range(NBUF):
            issue_idx(j, j)
        for j in range(NBUF):
            wait_idx(j, j)
            issue_gather(j)

        n_main = (NB_E - NBUF) // NBUF * NBUF

        @pl.loop(0, n_main, step=NBUF)
        def _(g):
            for j in range(NBUF):
                wait_gather(j)
                scale(j)
                issue_scatter(j)
            for j in range(NBUF):
                b2 = g + NBUF + j
                wait_scatter(j)
                issue_idx(b2, j)
            for j in range(NBUF):
                b2 = g + NBUF + j
                wait_idx(b2, j)
                issue_gather(j)

        for j in range(NBUF):
            wait_gather(j)
            scale(j)
            issue_scatter(j)
        for b in range(n_main + NBUF, NB_E):
            wait_scatter(0)
            issue_idx(b, 0)
            wait_idx(b, 0)
            issue_gather(0)
            wait_gather(0)
            scale(0)
            issue_scatter(0)
        for j in range(NBUF):
            wait_scatter(j)

        plsc.subcore_barrier()
        for t in range(ROWS_PT // ZROWS):
            pltpu.sync_copy(
                acc.at[pl.ds(s * ROWS_PT + t * ZROWS, ZROWS)],
                out_hbm.at[c, pl.ds(s * ROWS_PT + t * ZROWS, ZROWS)],
            )

    return _k


_BN = 400
_NBLK = N // _BN


def _dot(a, b):
    return lax.dot_general(
        a, b, (((1,), (0,)), ((), ())),
        preferred_element_type=jnp.float32,
        precision=lax.Precision.HIGHEST,
    )


def _tc_mm1disg(x, w1, d0, d1):
    def body(x_ref, w_ref, d0_ref, d1_ref, dis_ref, g_ref):
        h1 = _dot(x_ref[...], w_ref[...])
        dis = lax.rsqrt(1.0 + d0_ref[...] + d1_ref[...])
        dis_ref[...] = dis
        g_ref[...] = h1 * dis

    return pl.pallas_call(
        body,
        grid=(_NBLK,),
        in_specs=[
            pl.BlockSpec((_BN, F), lambda i: (i, 0)),
            pl.BlockSpec((F, HID), lambda i: (0, 0)),
            pl.BlockSpec((_BN, 1), lambda i: (i, 0)),
            pl.BlockSpec((_BN, 1), lambda i: (i, 0)),
        ],
        out_specs=[
            pl.BlockSpec((_BN, 1), lambda i: (i, 0)),
            pl.BlockSpec((_BN, HID), lambda i: (i, 0)),
        ],
        out_shape=[
            jax.ShapeDtypeStruct((N, 1), jnp.float32),
            jax.ShapeDtypeStruct((N, HID), jnp.float32),
        ],
    )(x, w1, d0, d1)


def _tc_mid(s1, g1, dis, b1, w2):
    def body(s_ref, g_ref, dis_ref, b_ref, w_ref, o_ref):
        ssum = s_ref[0] + s_ref[1] + g_ref[...]
        z = jax.nn.sigmoid(dis_ref[...] * ssum + b_ref[...])
        h2 = _dot(z, w_ref[...])
        o_ref[...] = h2 * dis_ref[...]

    return pl.pallas_call(
        body,
        grid=(_NBLK,),
        in_specs=[
            pl.BlockSpec((NC, _BN, HID), lambda i: (0, i, 0)),
            pl.BlockSpec((_BN, HID), lambda i: (i, 0)),
            pl.BlockSpec((_BN, 1), lambda i: (i, 0)),
            pl.BlockSpec((1, HID), lambda i: (0, 0)),
            pl.BlockSpec((HID, 128), lambda i: (0, 0)),
        ],
        out_specs=pl.BlockSpec((_BN, 128), lambda i: (i, 0)),
        out_shape=jax.ShapeDtypeStruct((N, 128), jnp.float32),
    )(s1, g1, dis, b1, w2)


def _tc_final(s2, g2, dis, b2):
    def body(s_ref, g_ref, dis_ref, b_ref, o_ref):
        ssum = (s_ref[0] + s_ref[1] + g_ref[...])[:, :CLS]
        o_ref[...] = jnp.tanh(dis_ref[...] * ssum + b_ref[...])

    return pl.pallas_call(
        body,
        grid=(_NBLK,),
        in_specs=[
            pl.BlockSpec((NC, _BN, 128), lambda i: (0, i, 0)),
            pl.BlockSpec((_BN, 128), lambda i: (i, 0)),
            pl.BlockSpec((_BN, 1), lambda i: (i, 0)),
            pl.BlockSpec((1, CLS), lambda i: (0, 0)),
        ],
        out_specs=pl.BlockSpec((_BN, CLS), lambda i: (i, 0)),
        out_shape=jax.ShapeDtypeStruct((N, CLS), jnp.float32),
    )(s2, g2, dis, b2)


def kernel(x, edge_index, edge_weight, W1, b1, W2, b2):
    row = edge_index[0].astype(jnp.int32)
    col = edge_index[1].astype(jnp.int32)
    ew = edge_weight.astype(jnp.float32)

    deg_p = _make_sc_deg()(col, ew)

    d0 = deg_p[0, :N].reshape(N, 1)
    d1 = deg_p[1, :N].reshape(N, 1)
    dis, g1 = _tc_mm1disg(x, W1, d0, d1)

    s1 = _make_sc_scatter(128, 128)(row, col, ew, g1)

    w2p = jnp.concatenate(
        [W2, jnp.zeros((HID, 128 - CLS), jnp.float32)], axis=1
    )
    g2 = _tc_mid(s1, g1, dis, b1.reshape(1, HID), w2p)

    s2 = _make_sc_scatter(128, 128)(row, col, ew, g2)

    return _tc_final(s2, g2, dis, b2.reshape(1, CLS))
